# baseline (speedup 1.0000x reference)
import jax
import jax.numpy as jnp
from jax.experimental import pallas as pl
from jax.experimental.pallas import tpu as pltpu

_N = 10000
_TILE = 400
_EPS = 1e-5


def _pass1_kernel(adj_ref, feat_ref, w1_ref, h_ref, cs_ref, cq_ref, abf_ref,
                  x1_ref):
    i = pl.program_id(0)

    @pl.when(i == 0)
    def _init():
        x1 = jnp.dot(feat_ref[...], w1_ref[...],
                     preferred_element_type=jnp.float32)
        x1_ref[...] = x1.astype(jnp.bfloat16)
        cs_ref[...] = jnp.zeros_like(cs_ref)
        cq_ref[...] = jnp.zeros_like(cq_ref)

    a = adj_ref[...]
    abf_ref[...] = a.astype(jnp.bfloat16)
    h = jax.lax.dot_general(
        a, x1_ref[...], (((1,), (0,)), ((), ())),
        precision=jax.lax.Precision.DEFAULT,
        preferred_element_type=jnp.float32)
    h = jnp.maximum(h, 0.0)
    h_ref[...] = h
    cs_ref[...] += jnp.sum(h, axis=0, keepdims=True)
    cq_ref[...] += jnp.sum(h * h, axis=0, keepdims=True)


def _pass2_kernel(abf_ref, h_ref, mean_ref, s_ref, beta_ref, w2_ref, out_ref,
                  z_ref):
    @pl.when(pl.program_id(0) == 0)
    def _make_z():
        xb = (h_ref[...] - mean_ref[...]) * s_ref[...] + beta_ref[...]
        z = jnp.dot(xb.astype(jnp.bfloat16), w2_ref[...],
                    preferred_element_type=jnp.float32)
        z_ref[...] = z.astype(jnp.bfloat16)

    out_ref[...] = jnp.dot(abf_ref[...], z_ref[...],
                           preferred_element_type=jnp.float32)


def kernel(feat, adj, W1, W2, gamma, beta):
    n, in_feat = feat.shape
    hid = W1.shape[1]
    out_feat = W2.shape[1]
    grid = n // _TILE

    h, col_sum, col_sq, abf = pl.pallas_call(
        _pass1_kernel,
        grid=(grid,),
        in_specs=[
            pl.BlockSpec((_TILE, n), lambda i: (i, 0)),
            pl.BlockSpec((n, in_feat), lambda i: (0, 0)),
            pl.BlockSpec((in_feat, hid), lambda i: (0, 0)),
        ],
        out_specs=[
            pl.BlockSpec((_TILE, hid), lambda i: (i, 0)),
            pl.BlockSpec((1, hid), lambda i: (0, 0)),
            pl.BlockSpec((1, hid), lambda i: (0, 0)),
            pl.BlockSpec((_TILE, n), lambda i: (i, 0)),
        ],
        out_shape=[
            jax.ShapeDtypeStruct((n, hid), jnp.float32),
            jax.ShapeDtypeStruct((1, hid), jnp.float32),
            jax.ShapeDtypeStruct((1, hid), jnp.float32),
            jax.ShapeDtypeStruct((n, n), jnp.bfloat16),
        ],
        scratch_shapes=[pltpu.VMEM((n, hid), jnp.bfloat16)],
        compiler_params=pltpu.CompilerParams(
            dimension_semantics=("arbitrary",),
        ),
    )(adj, feat.astype(jnp.bfloat16), W1.astype(jnp.bfloat16))

    mean = (col_sum[0] / n)[None, :]
    var = col_sq[0] / n - mean[0] * mean[0]
    s = (gamma * jax.lax.rsqrt(var + _EPS))[None, :]

    out = pl.pallas_call(
        _pass2_kernel,
        grid=(grid,),
        in_specs=[
            pl.BlockSpec((_TILE, n), lambda i: (i, 0)),
            pl.BlockSpec((n, hid), lambda i: (0, 0)),
            pl.BlockSpec((1, hid), lambda i: (0, 0)),
            pl.BlockSpec((1, hid), lambda i: (0, 0)),
            pl.BlockSpec((1, hid), lambda i: (0, 0)),
            pl.BlockSpec((hid, out_feat), lambda i: (0, 0)),
        ],
        out_specs=pl.BlockSpec((_TILE, out_feat), lambda i: (i, 0)),
        out_shape=jax.ShapeDtypeStruct((n, out_feat), jnp.float32),
        scratch_shapes=[pltpu.VMEM((n, out_feat), jnp.bfloat16)],
        compiler_params=pltpu.CompilerParams(
            dimension_semantics=("arbitrary",),
        ),
    )(abf, h, mean, s, beta[None, :], W2.astype(jnp.bfloat16))

    return out


# speedup vs baseline: 1.1112x; 1.1112x over previous
import jax
import jax.numpy as jnp
from jax.experimental import pallas as pl
from jax.experimental.pallas import tpu as pltpu

_N = 10000
_TILE = 400
_EPS = 1e-5


def _fused_kernel(adj_ref, feat_ref, w1_ref, w2_ref, g_ref, b_ref, out_ref,
                  h_ref, x1_ref, z_ref, cs_ref, cq_ref):
    p = pl.program_id(0)
    i = pl.program_id(1)
    n = h_ref.shape[0]

    @pl.when((p == 0) & (i == 0))
    def _init():
        x1 = jnp.dot(feat_ref[...], w1_ref[...],
                     preferred_element_type=jnp.float32)
        x1_ref[...] = x1.astype(jnp.bfloat16)
        cs_ref[...] = jnp.zeros_like(cs_ref)
        cq_ref[...] = jnp.zeros_like(cq_ref)

    @pl.when(p == 0)
    def _phase0():
        h = jax.lax.dot_general(
            adj_ref[...], x1_ref[...], (((1,), (0,)), ((), ())),
            precision=jax.lax.Precision.DEFAULT,
            preferred_element_type=jnp.float32)
        h = jnp.maximum(h, 0.0)
        h_ref[pl.ds(i * _TILE, _TILE), :] = h
        cs_ref[...] += jnp.sum(h, axis=0, keepdims=True)
        cq_ref[...] += jnp.sum(h * h, axis=0, keepdims=True)

    @pl.when((p == 1) & (i == 0))
    def _make_z():
        mean = cs_ref[...] / n
        var = cq_ref[...] / n - mean * mean
        s = g_ref[...] * jax.lax.rsqrt(var + _EPS)
        xb = (h_ref[...] - mean) * s + b_ref[...]
        z = jnp.dot(xb.astype(jnp.bfloat16), w2_ref[...],
                    preferred_element_type=jnp.float32)
        z_ref[...] = z.astype(jnp.bfloat16)

    @pl.when(p == 1)
    def _phase1():
        out_ref[...] = jax.lax.dot_general(
            adj_ref[...], z_ref[...], (((1,), (0,)), ((), ())),
            precision=jax.lax.Precision.DEFAULT,
            preferred_element_type=jnp.float32)


def kernel(feat, adj, W1, W2, gamma, beta):
    n, in_feat = feat.shape
    hid = W1.shape[1]
    out_feat = W2.shape[1]
    grid = n // _TILE

    out = pl.pallas_call(
        _fused_kernel,
        grid=(2, grid),
        in_specs=[
            pl.BlockSpec((_TILE, n), lambda p, i: (i, 0)),
            pl.BlockSpec((n, in_feat), lambda p, i: (0, 0)),
            pl.BlockSpec((in_feat, hid), lambda p, i: (0, 0)),
            pl.BlockSpec((hid, out_feat), lambda p, i: (0, 0)),
            pl.BlockSpec((1, hid), lambda p, i: (0, 0)),
            pl.BlockSpec((1, hid), lambda p, i: (0, 0)),
        ],
        out_specs=pl.BlockSpec((_TILE, out_feat), lambda p, i: (p * i, 0)),
        out_shape=jax.ShapeDtypeStruct((n, out_feat), jnp.float32),
        scratch_shapes=[
            pltpu.VMEM((n, hid), jnp.float32),
            pltpu.VMEM((n, hid), jnp.bfloat16),
            pltpu.VMEM((n, out_feat), jnp.bfloat16),
            pltpu.VMEM((1, hid), jnp.float32),
            pltpu.VMEM((1, hid), jnp.float32),
        ],
        compiler_params=pltpu.CompilerParams(
            dimension_semantics=("arbitrary", "arbitrary"),
        ),
    )(adj, feat.astype(jnp.bfloat16), W1.astype(jnp.bfloat16),
      W2.astype(jnp.bfloat16), gamma[None, :], beta[None, :])

    return out
